# Initial kernel scaffold; baseline (speedup 1.0000x reference)
import jax
import jax.numpy as jnp
from jax.experimental import pallas as pl


def kernel(rpn_reg_absolute, rpn_cls):
    raise NotImplementedError("write your pallas kernel here")


# speedup vs baseline: 409.2383x; 409.2383x over previous
import functools

import jax
import jax.numpy as jnp
from jax.experimental import pallas as pl
from jax.experimental.pallas import tpu as pltpu

_N_OUT = 1000
_IOU_THR = 0.7
_T = 128
_A = 20000


def _nms_body(c0_ref, c1_ref, c2_ref, c3_ref,
              keep_ref, cnt_ref,
              area_ref, s_ref, sup_ref, alive_ref, ident_ref, conv_ref):
    NB = c0_ref.shape[0]
    T = _T

    ia = jax.lax.broadcasted_iota(jnp.int32, (T, T), 0)
    ib = jax.lax.broadcasted_iota(jnp.int32, (T, T), 1)
    ident_ref[...] = jnp.where(ia == ib, 1.0, 0.0)
    cnt_ref[0, 0] = 0

    def init_body(bi, _):
        r0 = c0_ref[pl.ds(bi, 1)].reshape(1, T)
        r1 = c1_ref[pl.ds(bi, 1)].reshape(1, T)
        r2 = c2_ref[pl.ds(bi, 1)].reshape(1, T)
        r3 = c3_ref[pl.ds(bi, 1)].reshape(1, T)
        area_ref[pl.ds(bi, 1)] = ((r3 - r1) * (r2 - r0)).reshape(1, 1, T)
        keep_ref[pl.ds(bi, 1)] = jnp.zeros((1, 1, T), jnp.float32)
        return 0

    jax.lax.fori_loop(0, NB, init_body, 0)

    ident = ident_ref[...]

    def _to_col(row):
        return jax.lax.dot_general(ident, row, (((1,), (1,)), ((), ())),
                                   preferred_element_type=jnp.float32)

    def _iou_mask(cols, col_area, rows, row_area):
        s0, s1, s2, s3 = cols
        r0, r1, r2, r3 = rows
        xx1 = jnp.maximum(s1, r1)
        yy1 = jnp.maximum(s0, r0)
        xx2 = jnp.minimum(s3, r3)
        yy2 = jnp.minimum(s2, r2)
        w = jnp.maximum(xx2 - xx1, 0.0)
        h = jnp.maximum(yy2 - yy1, 0.0)
        inter = w * h
        iou = inter / (col_area + row_area - inter)
        return jnp.where(iou > _IOU_THR, 1.0, 0.0)

    def block_body(bi, _):
        @pl.when(cnt_ref[0, 0] < _N_OUT)
        def _():
            r0 = c0_ref[pl.ds(bi, 1)].reshape(1, T)
            r1 = c1_ref[pl.ds(bi, 1)].reshape(1, T)
            r2 = c2_ref[pl.ds(bi, 1)].reshape(1, T)
            r3 = c3_ref[pl.ds(bi, 1)].reshape(1, T)
            r_area = area_ref[pl.ds(bi, 1)].reshape(1, T)

            sup_ref[...] = jnp.zeros((1, T), jnp.float32)

            def cross_body(bj, _c):
                s_rows = (c0_ref[pl.ds(bj, 1)].reshape(1, T),
                          c1_ref[pl.ds(bj, 1)].reshape(1, T),
                          c2_ref[pl.ds(bj, 1)].reshape(1, T),
                          c3_ref[pl.ds(bj, 1)].reshape(1, T))
                cols = tuple(_to_col(r) for r in s_rows)
                c_area = _to_col(area_ref[pl.ds(bj, 1)].reshape(1, T))
                m = _iou_mask(cols, c_area, (r0, r1, r2, r3), r_area)
                kr = keep_ref[pl.ds(bj, 1)].reshape(1, T)
                dead = jax.lax.dot_general(kr, m, (((1,), (0,)), ((), ())),
                                           preferred_element_type=jnp.float32)
                sup_ref[...] = jnp.maximum(sup_ref[...],
                                           jnp.where(dead > 0.0, 1.0, 0.0))
                return 0

            jax.lax.fori_loop(0, bi, cross_body, 0)

            cols = (_to_col(r0), _to_col(r1), _to_col(r2), _to_col(r3))
            c_area = _to_col(r_area)
            m = _iou_mask(cols, c_area, (r0, r1, r2, r3), r_area)
            s_ref[...] = m * jnp.where(ia < ib, 1.0, 0.0)

            lane = jax.lax.broadcasted_iota(jnp.int32, (1, T), 1)
            valid = jnp.where(bi * T + lane < _A, 1.0, 0.0)
            alive0 = (1.0 - sup_ref[...]) * valid
            alive_ref[...] = alive0
            conv_ref[0] = 0

            def fix_body(t, _f):
                @pl.when(conv_ref[0] == 0)
                def _():
                    alive = alive_ref[...]
                    dead = jax.lax.dot_general(
                        alive, s_ref[...], (((1,), (0,)), ((), ())),
                        preferred_element_type=jnp.float32)
                    new_alive = alive0 * jnp.where(dead > 0.0, 0.0, 1.0)
                    changed = jnp.sum(jnp.abs(new_alive - alive))
                    alive_ref[...] = new_alive
                    conv_ref[0] = jnp.where(changed > 0.0, 0, 1)
                return 0

            jax.lax.fori_loop(0, T, fix_body, 0)

            alive = alive_ref[...]
            keep_ref[pl.ds(bi, 1)] = alive.reshape(1, 1, T)
            cnt_ref[0, 0] = cnt_ref[0, 0] + jnp.sum(alive).astype(jnp.int32)
        return 0

    jax.lax.fori_loop(0, NB, block_body, 0)


@functools.partial(jax.jit, static_argnums=())
def kernel(rpn_reg_absolute, rpn_cls):
    B, A = rpn_cls.shape
    boxes = rpn_reg_absolute[0]
    scores = rpn_cls[0]

    order = jnp.argsort(-jax.lax.stop_gradient(scores))
    boxes_s = boxes[order]
    scores_s = scores[order]

    NB = (A + _T - 1) // _T
    pad = NB * _T - A
    bp = jnp.pad(boxes_s, ((0, pad), (0, 0)))
    crows = [bp[:, i].reshape(NB, 1, _T) for i in range(4)]

    keep2d, cnt = pl.pallas_call(
        _nms_body,
        out_shape=[
            jax.ShapeDtypeStruct((NB, 1, _T), jnp.float32),
            jax.ShapeDtypeStruct((1, 1), jnp.int32),
        ],
        out_specs=[
            pl.BlockSpec(memory_space=pltpu.VMEM),
            pl.BlockSpec(memory_space=pltpu.SMEM),
        ],
        in_specs=[pl.BlockSpec(memory_space=pltpu.VMEM)] * 4,
        scratch_shapes=[
            pltpu.VMEM((NB, 1, _T), jnp.float32),
            pltpu.VMEM((_T, _T), jnp.float32),
            pltpu.VMEM((1, _T), jnp.float32),
            pltpu.VMEM((1, _T), jnp.float32),
            pltpu.VMEM((_T, _T), jnp.float32),
            pltpu.SMEM((1,), jnp.int32),
        ],
    )(*crows)

    keep = keep2d.reshape(NB * _T)[:A] > 0.5
    L = cnt[0, 0]

    keep_i = keep.astype(jnp.int32)
    ranks = jnp.cumsum(keep_i) - keep_i
    slot = jnp.where(keep & (ranks < _N_OUT), ranks, _N_OUT)
    pos = jnp.zeros((_N_OUT + 1,), jnp.int32).at[slot].set(
        jnp.arange(A, dtype=jnp.int32), mode="drop")
    j = jnp.arange(_N_OUT, dtype=jnp.int32)
    sel = jnp.where(j < L, pos[:_N_OUT], jnp.maximum(j - L, 0))

    out_boxes = boxes_s[sel][None]
    out_scores = scores_s[sel][None]
    return (out_boxes, out_scores)


# speedup vs baseline: 710.4868x; 1.7361x over previous
import functools

import jax
import jax.numpy as jnp
from jax import lax
from jax.experimental import pallas as pl
from jax.experimental.pallas import tpu as pltpu
from jax.experimental.pallas import tpu_sc as plsc

_N_OUT = 1000
_IOU_THR = 0.7
_T = 128
_A = 20000
_NB = (_A + _T - 1) // _T
_NBT = _NB * _T
_NV = _NBT // 16
_SEL_PAD = 1024
_NW = 32
_GPAD = 20480
_PER_W = _GPAD // _NW
_CHUNK = 128
_NCH = _PER_W // _CHUNK


def _nms_body(c0_ref, c1_ref, c2_ref, c3_ref,
              keep_ref, cnt_ref,
              area_ref, s_ref, sup_ref, alive_ref, ident_ref, conv_ref):
    NB = c0_ref.shape[0]
    T = _T

    ia = jax.lax.broadcasted_iota(jnp.int32, (T, T), 0)
    ib = jax.lax.broadcasted_iota(jnp.int32, (T, T), 1)
    ident_ref[...] = jnp.where(ia == ib, 1.0, 0.0)
    cnt_ref[0, 0] = 0

    def init_body(bi, _):
        r0 = c0_ref[pl.ds(bi, 1)].reshape(1, T)
        r1 = c1_ref[pl.ds(bi, 1)].reshape(1, T)
        r2 = c2_ref[pl.ds(bi, 1)].reshape(1, T)
        r3 = c3_ref[pl.ds(bi, 1)].reshape(1, T)
        area_ref[pl.ds(bi, 1)] = ((r3 - r1) * (r2 - r0)).reshape(1, 1, T)
        keep_ref[pl.ds(bi, 1)] = jnp.zeros((1, 1, T), jnp.float32)
        return 0

    jax.lax.fori_loop(0, NB, init_body, 0)

    ident = ident_ref[...]

    def _to_col(row):
        return jax.lax.dot_general(ident, row, (((1,), (1,)), ((), ())),
                                   preferred_element_type=jnp.float32)

    def _iou_mask(cols, col_area, rows, row_area):
        s0, s1, s2, s3 = cols
        r0, r1, r2, r3 = rows
        xx1 = jnp.maximum(s1, r1)
        yy1 = jnp.maximum(s0, r0)
        xx2 = jnp.minimum(s3, r3)
        yy2 = jnp.minimum(s2, r2)
        w = jnp.maximum(xx2 - xx1, 0.0)
        h = jnp.maximum(yy2 - yy1, 0.0)
        inter = w * h
        iou = inter / (col_area + row_area - inter)
        return jnp.where(iou > _IOU_THR, 1.0, 0.0)

    def block_body(bi, _):
        @pl.when(cnt_ref[0, 0] < _N_OUT)
        def _():
            r0 = c0_ref[pl.ds(bi, 1)].reshape(1, T)
            r1 = c1_ref[pl.ds(bi, 1)].reshape(1, T)
            r2 = c2_ref[pl.ds(bi, 1)].reshape(1, T)
            r3 = c3_ref[pl.ds(bi, 1)].reshape(1, T)
            r_area = area_ref[pl.ds(bi, 1)].reshape(1, T)

            sup_ref[...] = jnp.zeros((1, T), jnp.float32)

            def cross_body(bj, _c):
                s_rows = (c0_ref[pl.ds(bj, 1)].reshape(1, T),
                          c1_ref[pl.ds(bj, 1)].reshape(1, T),
                          c2_ref[pl.ds(bj, 1)].reshape(1, T),
                          c3_ref[pl.ds(bj, 1)].reshape(1, T))
                cols = tuple(_to_col(r) for r in s_rows)
                c_area = _to_col(area_ref[pl.ds(bj, 1)].reshape(1, T))
                m = _iou_mask(cols, c_area, (r0, r1, r2, r3), r_area)
                kr = keep_ref[pl.ds(bj, 1)].reshape(1, T)
                dead = jax.lax.dot_general(kr, m, (((1,), (0,)), ((), ())),
                                           preferred_element_type=jnp.float32)
                sup_ref[...] = jnp.maximum(sup_ref[...],
                                           jnp.where(dead > 0.0, 1.0, 0.0))
                return 0

            jax.lax.fori_loop(0, bi, cross_body, 0)

            cols = (_to_col(r0), _to_col(r1), _to_col(r2), _to_col(r3))
            c_area = _to_col(r_area)
            m = _iou_mask(cols, c_area, (r0, r1, r2, r3), r_area)
            s_ref[...] = m * jnp.where(ia < ib, 1.0, 0.0)

            lane = jax.lax.broadcasted_iota(jnp.int32, (1, T), 1)
            valid = jnp.where(bi * T + lane < _A, 1.0, 0.0)
            alive0 = (1.0 - sup_ref[...]) * valid
            alive_ref[...] = alive0
            conv_ref[0] = 0

            def fix_body(t, _f):
                @pl.when(conv_ref[0] == 0)
                def _():
                    alive = alive_ref[...]
                    dead = jax.lax.dot_general(
                        alive, s_ref[...], (((1,), (0,)), ((), ())),
                        preferred_element_type=jnp.float32)
                    new_alive = alive0 * jnp.where(dead > 0.0, 0.0, 1.0)
                    changed = jnp.sum(jnp.abs(new_alive - alive))
                    alive_ref[...] = new_alive
                    conv_ref[0] = jnp.where(changed > 0.0, 0, 1)
                return 0

            jax.lax.fori_loop(0, T, fix_body, 0)

            alive = alive_ref[...]
            keep_ref[pl.ds(bi, 1)] = alive.reshape(1, 1, T)
            cnt_ref[0, 0] = cnt_ref[0, 0] + jnp.sum(alive).astype(jnp.int32)
        return 0

    jax.lax.fori_loop(0, NB, block_body, 0)


def _sc_select_body(keep_hbm, cnt_hbm, c0_hbm, c1_hbm, c2_hbm, c3_hbm, sc_hbm,
                    o0_hbm, o1_hbm, o2_hbm, o3_hbm, os_hbm,
                    keep_v, cnt_v, c0_v, c1_v, c2_v, c3_v, sc_v,
                    sel_v, o0_v, o1_v, o2_v, o3_v, os_v):
    @pl.when(jnp.logical_and(lax.axis_index("c") == 0, lax.axis_index("s") == 0))
    def _():
        pltpu.sync_copy(keep_hbm, keep_v)
        pltpu.sync_copy(cnt_hbm, cnt_v)
        pltpu.sync_copy(c0_hbm, c0_v)
        pltpu.sync_copy(c1_hbm, c1_v)
        pltpu.sync_copy(c2_hbm, c2_v)
        pltpu.sync_copy(c3_hbm, c3_v)
        pltpu.sync_copy(sc_hbm, sc_v)

        iota = lax.iota(jnp.int32, 16)
        lv = cnt_v[...]

        def init_body(j, _):
            g = j * 16 + iota
            sel_v[pl.ds(j * 16, 16)] = jnp.maximum(g - lv, 0)
            return 0

        lax.fori_loop(0, _SEL_PAD // 16, init_body, 0)

        def body(i, total):
            k = keep_v[pl.ds(i * 16, 16)]
            incl = plsc.cumsum(k)
            rank = incl - k + total
            mask = jnp.logical_and(k > 0, rank < _N_OUT)
            plsc.store_scatter(sel_v, [rank], i * 16 + iota, mask=mask)
            return total + jnp.sum(k)

        lax.fori_loop(0, _NV, body, jnp.int32(0))

        def gather_body(j, _):
            idx = sel_v[pl.ds(j * 16, 16)]
            o0_v[pl.ds(j * 16, 16)] = plsc.load_gather(c0_v, [idx])
            o1_v[pl.ds(j * 16, 16)] = plsc.load_gather(c1_v, [idx])
            o2_v[pl.ds(j * 16, 16)] = plsc.load_gather(c2_v, [idx])
            o3_v[pl.ds(j * 16, 16)] = plsc.load_gather(c3_v, [idx])
            os_v[pl.ds(j * 16, 16)] = plsc.load_gather(sc_v, [idx])
            return 0

        lax.fori_loop(0, _SEL_PAD // 16, gather_body, 0)

        pltpu.sync_copy(o0_v, o0_hbm)
        pltpu.sync_copy(o1_v, o1_hbm)
        pltpu.sync_copy(o2_v, o2_hbm)
        pltpu.sync_copy(o3_v, o3_hbm)
        pltpu.sync_copy(os_v, os_hbm)


def _sc_select(keep_i, cnt_vec, c0, c1, c2, c3, sc):
    f32, i32 = jnp.float32, jnp.int32
    k = pl.kernel(
        _sc_select_body,
        mesh=plsc.VectorSubcoreMesh(core_axis_name="c", subcore_axis_name="s"),
        compiler_params=pltpu.CompilerParams(needs_layout_passes=False),
        out_type=[jax.ShapeDtypeStruct((_SEL_PAD,), f32)] * 5,
        scratch_types=[
            pltpu.VMEM((_NBT,), i32),
            pltpu.VMEM((16,), i32),
            pltpu.VMEM((_NBT,), f32),
            pltpu.VMEM((_NBT,), f32),
            pltpu.VMEM((_NBT,), f32),
            pltpu.VMEM((_NBT,), f32),
            pltpu.VMEM((_NBT,), f32),
            pltpu.VMEM((_SEL_PAD,), i32),
            pltpu.VMEM((_SEL_PAD,), f32),
            pltpu.VMEM((_SEL_PAD,), f32),
            pltpu.VMEM((_SEL_PAD,), f32),
            pltpu.VMEM((_SEL_PAD,), f32),
            pltpu.VMEM((_SEL_PAD,), f32),
        ],
    )
    return k(keep_i, cnt_vec, c0, c1, c2, c3, sc)


def _sc_pregather_body(order_hbm, b0_hbm, b1_hbm, b2_hbm, b3_hbm, sc_hbm,
                       o0_hbm, o1_hbm, o2_hbm, o3_hbm, os_hbm,
                       idx_v, g_v, sem):
    wid = lax.axis_index("s") * 2 + lax.axis_index("c")
    base = wid * _PER_W
    pltpu.sync_copy(order_hbm.at[pl.ds(base, _PER_W)], idx_v)
    srcs = (b0_hbm, b1_hbm, b2_hbm, b3_hbm, sc_hbm)
    dsts = (o0_hbm, o1_hbm, o2_hbm, o3_hbm, os_hbm)
    for a in range(5):
        for ch in range(_NCH):
            pltpu.async_copy(
                srcs[a].at[idx_v.at[pl.ds(ch * _CHUNK, _CHUNK)]],
                g_v.at[pl.ds(ch * _CHUNK, _CHUNK)], sem).wait()
        pltpu.sync_copy(g_v, dsts[a].at[pl.ds(base, _PER_W)])


def _sc_pregather(order_pad, b0, b1, b2, b3, sc):
    f32, i32 = jnp.float32, jnp.int32
    k = pl.kernel(
        _sc_pregather_body,
        mesh=plsc.VectorSubcoreMesh(core_axis_name="c", subcore_axis_name="s"),
        compiler_params=pltpu.CompilerParams(needs_layout_passes=False),
        out_type=[jax.ShapeDtypeStruct((_GPAD,), f32)] * 5,
        scratch_types=[
            pltpu.VMEM((_PER_W,), i32),
            pltpu.VMEM((_PER_W,), f32),
            pltpu.SemaphoreType.DMA,
        ],
    )
    return k(order_pad, b0, b1, b2, b3, sc)


@functools.partial(jax.jit, static_argnums=())
def kernel(rpn_reg_absolute, rpn_cls):
    B, A = rpn_cls.shape
    boxes = rpn_reg_absolute[0]
    scores = rpn_cls[0]

    order = jnp.argsort(-jax.lax.stop_gradient(scores))
    order_pad = jnp.pad(order.astype(jnp.int32), (0, _GPAD - A))

    c0s, c1s, c2s, c3s, scs = _sc_pregather(
        order_pad, boxes[:, 0], boxes[:, 1], boxes[:, 2], boxes[:, 3], scores)

    NB = _NB
    crows = [c[:_NBT].reshape(NB, 1, _T) for c in (c0s, c1s, c2s, c3s)]

    keep2d, cnt = pl.pallas_call(
        _nms_body,
        out_shape=[
            jax.ShapeDtypeStruct((NB, 1, _T), jnp.float32),
            jax.ShapeDtypeStruct((1, 1), jnp.int32),
        ],
        out_specs=[
            pl.BlockSpec(memory_space=pltpu.VMEM),
            pl.BlockSpec(memory_space=pltpu.SMEM),
        ],
        in_specs=[pl.BlockSpec(memory_space=pltpu.VMEM)] * 4,
        scratch_shapes=[
            pltpu.VMEM((NB, 1, _T), jnp.float32),
            pltpu.VMEM((_T, _T), jnp.float32),
            pltpu.VMEM((1, _T), jnp.float32),
            pltpu.VMEM((1, _T), jnp.float32),
            pltpu.VMEM((_T, _T), jnp.float32),
            pltpu.SMEM((1,), jnp.int32),
        ],
    )(*crows)

    keep_i = (keep2d.reshape(_NBT) > 0.5).astype(jnp.int32)
    cnt_vec = jnp.full((16,), cnt[0, 0], jnp.int32)

    o0, o1, o2, o3, osc = _sc_select(
        keep_i, cnt_vec, c0s[:_NBT], c1s[:_NBT], c2s[:_NBT], c3s[:_NBT],
        scs[:_NBT])

    out_boxes = jnp.stack(
        [o0[:_N_OUT], o1[:_N_OUT], o2[:_N_OUT], o3[:_N_OUT]], axis=-1)[None]
    out_scores = osc[:_N_OUT][None]
    return (out_boxes, out_scores)
